# baseline (speedup 1.0000x reference)
import jax
import jax.numpy as jnp
from jax.experimental import pallas as pl
from jax.experimental.pallas import tpu as pltpu


DIM = 64
N_CODES = 1024
PIX = 1024


def _vq_body(x_ref, embed_ref, quant_ref, diff_ref, ind_ref,
             esq_ref, iota_ref, embbf_ref):
    @pl.when(pl.program_id(0) == 0)
    def _init():
        emb0 = embed_ref[...]
        embt = jnp.transpose(emb0)
        esq_ref[...] = jnp.sum(embt * embt, axis=1, keepdims=True)
        iota_ref[...] = jax.lax.broadcasted_iota(
            jnp.int32, (N_CODES, PIX), 0)
        embbf_ref[...] = emb0.astype(jnp.bfloat16)

    x = x_ref[0]
    embed = embed_ref[...]
    esq = esq_ref[...]

    scores2 = jax.lax.dot_general(
        embed, -2.0 * x, (((0,), (0,)), ((), ())),
        preferred_element_type=jnp.float32)
    xsq = jnp.sum(x * x, axis=0, keepdims=True)
    dist = (xsq + scores2) + esq

    ind = jnp.argmin(dist, axis=0, keepdims=True)
    ind_ref[0] = ind

    onehot = (iota_ref[...] == ind).astype(jnp.bfloat16)
    quant = jax.lax.dot_general(
        embbf_ref[...], onehot, (((1,), (0,)), ((), ())),
        preferred_element_type=jnp.float32)
    quant_ref[0] = quant
    diff_ref[0] = quant - x


def kernel(inputs, embed):
    b, c, h, w = inputs.shape
    x = inputs.reshape(b, c, h * w)

    quant, diff, ind = pl.pallas_call(
        _vq_body,
        grid=(b,),
        in_specs=[
            pl.BlockSpec((1, c, h * w), lambda i: (i, 0, 0)),
            pl.BlockSpec((DIM, N_CODES), lambda i: (0, 0)),
        ],
        out_specs=[
            pl.BlockSpec((1, c, h * w), lambda i: (i, 0, 0)),
            pl.BlockSpec((1, c, h * w), lambda i: (i, 0, 0)),
            pl.BlockSpec((1, 1, h * w), lambda i: (i, 0, 0)),
        ],
        out_shape=[
            jax.ShapeDtypeStruct((b, c, h * w), jnp.float32),
            jax.ShapeDtypeStruct((b, c, h * w), jnp.float32),
            jax.ShapeDtypeStruct((b, 1, h * w), jnp.int32),
        ],
        scratch_shapes=[
            pltpu.VMEM((N_CODES, 1), jnp.float32),
            pltpu.VMEM((N_CODES, PIX), jnp.int32),
            pltpu.VMEM((DIM, N_CODES), jnp.bfloat16),
        ],
    )(x, embed)

    return (quant.reshape(b, c, h, w),
            diff.reshape(b, c, h, w),
            ind.reshape(b, h, w))


# speedup vs baseline: 1.0649x; 1.0649x over previous
import jax
import jax.numpy as jnp
from jax.experimental import pallas as pl


DIM = 64
N_CODES = 1024
PIX = 1024
BPR = 2


def _vq_body(x_ref, embed_ref, quant_ref, diff_ref, ind_ref):
    embed = embed_ref[...]
    embt = jnp.transpose(embed)
    esq = jnp.sum(embt * embt, axis=1, keepdims=True)
    embbf = embed.astype(jnp.bfloat16)
    code_iota = jax.lax.broadcasted_iota(jnp.int32, (N_CODES, PIX), 0)

    for j in range(BPR):
        x = x_ref[j]

        scores2 = jax.lax.dot_general(
            embed, -2.0 * x, (((0,), (0,)), ((), ())),
            preferred_element_type=jnp.float32)
        xsq = jnp.sum(x * x, axis=0, keepdims=True)
        dist = (xsq + scores2) + esq

        ind = jnp.argmin(dist, axis=0, keepdims=True)
        ind_ref[j] = ind

        onehot = (code_iota == ind).astype(jnp.bfloat16)
        quant = jax.lax.dot_general(
            embbf, onehot, (((1,), (0,)), ((), ())),
            preferred_element_type=jnp.float32)
        quant_ref[j] = quant
        diff_ref[j] = quant - x


def kernel(inputs, embed):
    b, c, h, w = inputs.shape
    x = inputs.reshape(b, c, h * w)

    quant, diff, ind = pl.pallas_call(
        _vq_body,
        grid=(b // BPR,),
        in_specs=[
            pl.BlockSpec((BPR, c, h * w), lambda i: (i, 0, 0)),
            pl.BlockSpec((DIM, N_CODES), lambda i: (0, 0)),
        ],
        out_specs=[
            pl.BlockSpec((BPR, c, h * w), lambda i: (i, 0, 0)),
            pl.BlockSpec((BPR, c, h * w), lambda i: (i, 0, 0)),
            pl.BlockSpec((BPR, 1, h * w), lambda i: (i, 0, 0)),
        ],
        out_shape=[
            jax.ShapeDtypeStruct((b, c, h * w), jnp.float32),
            jax.ShapeDtypeStruct((b, c, h * w), jnp.float32),
            jax.ShapeDtypeStruct((b, 1, h * w), jnp.int32),
        ],
    )(x, embed)

    return (quant.reshape(b, c, h, w),
            diff.reshape(b, c, h, w),
            ind.reshape(b, h, w))


# speedup vs baseline: 1.0882x; 1.0219x over previous
import jax
import jax.numpy as jnp
from jax.experimental import pallas as pl


DIM = 64
N_CODES = 1024
PIX = 1024
BPR = 4


def _vq_body(x_ref, embed_ref, quant_ref, diff_ref, ind_ref):
    embed = embed_ref[...]
    embt = jnp.transpose(embed)
    esq = jnp.sum(embt * embt, axis=1, keepdims=True)
    embbf = embed.astype(jnp.bfloat16)
    code_iota = jax.lax.broadcasted_iota(jnp.int32, (N_CODES, PIX), 0)

    for j in range(BPR):
        x = x_ref[j]

        scores2 = jax.lax.dot_general(
            embed, -2.0 * x, (((0,), (0,)), ((), ())),
            preferred_element_type=jnp.float32)
        xsq = jnp.sum(x * x, axis=0, keepdims=True)
        dist = (xsq + scores2) + esq

        ind = jnp.argmin(dist, axis=0, keepdims=True)
        ind_ref[j] = ind

        onehot = (code_iota == ind).astype(jnp.bfloat16)
        quant = jax.lax.dot_general(
            embbf, onehot, (((1,), (0,)), ((), ())),
            preferred_element_type=jnp.float32)
        quant_ref[j] = quant
        diff_ref[j] = quant - x


def kernel(inputs, embed):
    b, c, h, w = inputs.shape
    x = inputs.reshape(b, c, h * w)

    quant, diff, ind = pl.pallas_call(
        _vq_body,
        grid=(b // BPR,),
        in_specs=[
            pl.BlockSpec((BPR, c, h * w), lambda i: (i, 0, 0)),
            pl.BlockSpec((DIM, N_CODES), lambda i: (0, 0)),
        ],
        out_specs=[
            pl.BlockSpec((BPR, c, h * w), lambda i: (i, 0, 0)),
            pl.BlockSpec((BPR, c, h * w), lambda i: (i, 0, 0)),
            pl.BlockSpec((BPR, 1, h * w), lambda i: (i, 0, 0)),
        ],
        out_shape=[
            jax.ShapeDtypeStruct((b, c, h * w), jnp.float32),
            jax.ShapeDtypeStruct((b, c, h * w), jnp.float32),
            jax.ShapeDtypeStruct((b, 1, h * w), jnp.int32),
        ],
    )(x, embed)

    return (quant.reshape(b, c, h, w),
            diff.reshape(b, c, h, w),
            ind.reshape(b, h, w))
